# Initial kernel scaffold; baseline (speedup 1.0000x reference)
import jax
import jax.numpy as jnp
from jax.experimental import pallas as pl


def kernel(inputs, params):
    raise NotImplementedError("write your pallas kernel here")


# speedup vs baseline: 4.4584x; 4.4584x over previous
import jax
import jax.numpy as jnp
from jax import lax
from jax.experimental import pallas as pl
from jax.experimental.pallas import tpu as pltpu

N_RADIAL = 5
N_ANGULAR = 8
RA = N_RADIAL * N_ANGULAR
TEMPLATE_RADIUS = 0.75
K_LRF = 16
ISC_DIMS = (32, 64, 128)
N_CLASSES = 40
B, N = 4, 1024
NT = 4
TIL = N // NT
CMAX = 128

_HI = jax.lax.Precision.HIGHEST


def _dot(a, b):
  return jnp.dot(a, b, precision=_HI, preferred_element_type=jnp.float32)


def _dotd(a, b):
  return jnp.dot(a, b, preferred_element_type=jnp.float32)


def _rowmin_idx(x, iota, axis):
  m = jnp.min(x, axis=axis, keepdims=True)
  return jnp.min(jnp.where(x == m, iota, jnp.int32(1 << 30)), axis=axis)


def _geom_kernel(co_ref, nidx_ref, nbrs_ref):
  t = pl.program_id(1)
  co = co_ref[0]
  co_tile = co_ref[0, pl.ds(t * TIL, TIL), :]
  iota = lax.broadcasted_iota(jnp.int32, (TIL, N), 1)
  d2 = jnp.zeros((TIL, N), jnp.float32)
  for c in range(3):
    cc = jnp.reshape(co_tile[:, c], (TIL, 1))
    cr = jnp.reshape(co[:, c], (1, N))
    d2 = d2 + jnp.square(cc - cr)
  work = d2
  for j in range(K_LRF):
    amin = _rowmin_idx(work, iota, 1)
    sel = iota == amin[:, None]
    nbr = _dot(sel.astype(jnp.float32), co)
    nidx_ref[0, j:j + 1, :] = jnp.reshape(amin, (1, TIL))
    for c in range(3):
      nbrs_ref[0, j * 3 + c:j * 3 + c + 1, :] = jnp.reshape(nbr[:, c], (1, TIL))
    work = jnp.where(sel, jnp.float32(jnp.inf), work)


def _frames_kernel(put_ref, pvt_ref, tu_ref, tv_ref, m_ref):
  put = put_ref[0]
  pvt = pvt_ref[0]
  iota = lax.broadcasted_iota(jnp.int32, (K_LRF, N), 0)

  def body(ra, _):
    tu = jnp.reshape(tu_ref[ra], (1, 1))
    tv = jnp.reshape(tv_ref[ra], (1, 1))
    d2t = jnp.square(tu - put) + jnp.square(tv - pvt)
    work = d2t
    sels = []
    ws = []
    for j in range(3):
      amin = _rowmin_idx(work, iota, 0)
      sel = iota == amin[None, :]
      dmin = jnp.min(work, axis=0, keepdims=True)
      d3 = jnp.sqrt(jnp.maximum(dmin, 0.0) + jnp.float32(1e-8))
      ws.append(1.0 / (d3 + jnp.float32(1e-8)))
      sels.append(sel)
      work = jnp.where(sel, jnp.float32(jnp.inf), work)
    wsum = ws[0] + ws[1] + ws[2]
    m = jnp.zeros((K_LRF, N), jnp.float32)
    for j in range(3):
      m = m + sels[j].astype(jnp.float32) * (ws[j] / wsum)
    m_ref[0, ra] = m
    return 0

  lax.fori_loop(0, RA, body, 0)


def _net_kernel(sig0_ref, nidx_ref, m_ref,
                w10, b10, w20, b20, ws0,
                w11, b11, w21, b21, ws1,
                w12, b12, w22, b22, ws2,
                covp_ref, cur_ref, nb_ref, acc_ref):
  iota = lax.broadcasted_iota(jnp.int32, (N, N), 0)

  def elu(x):
    return jnp.where(x > 0, x, jnp.exp(jnp.minimum(x, 0.0)) - 1.0)

  def conv(x, cin, t, wf_ref, b_ref):
    for k in range(K_LRF):
      oht = (iota == nidx_ref[0, k, :]).astype(jnp.float32)
      nb_ref[k, :cin, :] = _dot(x, oht)
    acc_ref[...] = jnp.zeros((CMAX, N), jnp.float32)

    def body(ra, _):
      mg = m_ref[0, ra]
      it = mg[0:1, :] * nb_ref[0, :cin, :]
      for k in range(1, K_LRF):
        it = it + mg[k:k + 1, :] * nb_ref[k, :cin, :]
      acc_ref[:t, :] += _dotd(wf_ref[ra], it)
      return 0

    lax.fori_loop(0, RA, body, 0)
    return acc_ref[:t, :] + b_ref[...]

  cur_ref[0:3, :] = sig0_ref[0]
  blocks = ((3, 32, w10, b10, w20, b20, ws0),
            (32, 64, w11, b11, w21, b21, ws1),
            (64, 128, w12, b12, w22, b22, ws2))
  for cin, t, w1, b1, w2, b2, wsk in blocks:
    x = cur_ref[:cin, :]
    h1 = elu(conv(x, cin, t, w1, b1))
    h2 = conv(h1, t, t, w2, b2)
    skip = _dotd(wsk[...], x)
    cur_ref[:t, :] = elu(h2 + skip)

  sig = cur_ref[...]
  mu = jnp.mean(sig, axis=1, keepdims=True)
  xc = sig - mu
  covp = lax.dot_general(xc, xc, (((1,), (1,)), ((), ())),
                         preferred_element_type=jnp.float32)
  covp_ref[0] = covp / jnp.float32(N)


def _clf_kernel(flat_ref, wc_ref, bc_ref, out_ref):
  out_ref[...] = _dotd(flat_ref[...], wc_ref[...]) + bc_ref[0:1, :]


@jax.jit
def kernel(inputs, params):
  f32 = jnp.float32
  bt3 = lambda b, t: (b, 0, t)

  co = inputs - jnp.mean(inputs, axis=1, keepdims=True)

  nidx, nbrs_t = pl.pallas_call(
      _geom_kernel,
      grid=(B, NT),
      in_specs=[pl.BlockSpec((1, N, 3), lambda b, t: (b, 0, 0))],
      out_specs=[
          pl.BlockSpec((1, K_LRF, TIL), bt3),
          pl.BlockSpec((1, 3 * K_LRF, TIL), bt3),
      ],
      out_shape=[
          jax.ShapeDtypeStruct((B, K_LRF, N), jnp.int32),
          jax.ShapeDtypeStruct((B, 3 * K_LRF, N), f32),
      ],
  )(co)

  nbrs = jnp.transpose(nbrs_t.reshape(B, K_LRF, 3, N), (0, 3, 1, 2))
  mu = jnp.mean(nbrs, axis=2, keepdims=True)
  cen = nbrs - mu
  cov = jnp.einsum('bnki,bnkj->bnij', cen, cen) / K_LRF
  _, v = jnp.linalg.eigh(cov)
  normal = v[..., 0]
  t1 = v[..., 2]
  t2 = v[..., 1]
  sgn = jnp.sign(jnp.sum(normal * co, axis=-1, keepdims=True) + 1e-9)
  signal = normal * sgn
  diff = nbrs - co[:, :, None, :]
  pu = jnp.einsum('bnkc,bnc->bnk', diff, t1)
  pv = jnp.einsum('bnkc,bnc->bnk', diff, t2)
  put = jnp.transpose(pu, (0, 2, 1))
  pvt = jnp.transpose(pv, (0, 2, 1))
  sig0 = jnp.transpose(signal, (0, 2, 1))

  radii = TEMPLATE_RADIUS * (
      jnp.arange(1, N_RADIAL + 1, dtype=f32) / N_RADIAL)
  ang = 2.0 * jnp.pi * jnp.arange(N_ANGULAR, dtype=f32) / N_ANGULAR
  tu = (radii[:, None] * jnp.cos(ang)[None, :]).reshape(RA, 1)
  tv = (radii[:, None] * jnp.sin(ang)[None, :]).reshape(RA, 1)

  bmap3 = lambda b: (b, 0, 0)
  bmap4 = lambda b: (b, 0, 0, 0)
  zmap2 = lambda b: (0, 0)
  zmap3 = lambda b: (0, 0, 0)

  m = pl.pallas_call(
      _frames_kernel,
      grid=(B,),
      in_specs=[
          pl.BlockSpec((1, K_LRF, N), bmap3),
          pl.BlockSpec((1, K_LRF, N), bmap3),
          pl.BlockSpec((RA, 1), zmap2),
          pl.BlockSpec((RA, 1), zmap2),
      ],
      out_specs=pl.BlockSpec((1, RA, K_LRF, N), bmap4),
      out_shape=jax.ShapeDtypeStruct((B, RA, K_LRF, N), f32),
  )(put, pvt, tu, tv)

  pin = []
  cin = 3
  for i, dim in enumerate(ISC_DIMS):
    w1 = jnp.transpose(params['W1_%d' % i], (1, 2, 0, 3)).reshape(RA, dim, cin)
    w2 = jnp.transpose(params['W2_%d' % i], (1, 2, 0, 3)).reshape(RA, dim, dim)
    pin += [w1, params['b1_%d' % i].reshape(dim, 1),
            w2, params['b2_%d' % i].reshape(dim, 1),
            params['Ws_%d' % i].T]
    cin = dim

  pspecs = [pl.BlockSpec(p.shape, zmap3 if p.ndim == 3 else zmap2)
            for p in pin]

  covp = pl.pallas_call(
      _net_kernel,
      grid=(B,),
      in_specs=[
          pl.BlockSpec((1, 3, N), bmap3),
          pl.BlockSpec((1, K_LRF, N), bmap3),
          pl.BlockSpec((1, RA, K_LRF, N), bmap4),
      ] + pspecs,
      out_specs=pl.BlockSpec((1, CMAX, CMAX), bmap3),
      out_shape=jax.ShapeDtypeStruct((B, CMAX, CMAX), f32),
      scratch_shapes=[
          pltpu.VMEM((CMAX, N), f32),
          pltpu.VMEM((K_LRF, CMAX, N), f32),
          pltpu.VMEM((CMAX, N), f32),
      ],
  )(sig0, nidx, m, *pin)

  flat = covp.reshape(B, CMAX * CMAX)
  out = pl.pallas_call(
      _clf_kernel,
      in_specs=[
          pl.BlockSpec(flat.shape, None),
          pl.BlockSpec(params['Wc'].shape, None),
          pl.BlockSpec((1, N_CLASSES), None),
      ],
      out_shape=jax.ShapeDtypeStruct((B, N_CLASSES), f32),
  )(flat, params['Wc'], params['bc'].reshape(1, N_CLASSES))
  return out
